# Initial kernel scaffold; baseline (speedup 1.0000x reference)
import jax
import jax.numpy as jnp
from jax.experimental import pallas as pl


def kernel(x, W_qkv, W_out, b_out, pos_embedding):
    raise NotImplementedError("write your pallas kernel here")


# speedup vs baseline: 2.3462x; 2.3462x over previous
import jax
import jax.numpy as jnp
from jax.experimental import pallas as pl
from jax.experimental.pallas import tpu as pltpu

DIM = 1024
HEADS = 16
HEAD_DIM = 64
WINDOW = 64
INNER = HEADS * HEAD_DIM
SCALE = HEAD_DIM ** (-0.5)
SEQ = 4096
NWIN = SEQ // WINDOW
J = 8
T = J * WINDOW


def _fused_kernel(x_ref, wqkv_ref, wout_ref, bout_ref, pos_ref, o_ref):
    xb = x_ref[0].reshape(T, DIM)

    qkv = jnp.dot(xb, wqkv_ref[...], preferred_element_type=jnp.float32)

    ii = jax.lax.broadcasted_iota(jnp.int32, (WINDOW, WINDOW), 0)
    jj = jax.lax.broadcasted_iota(jnp.int32, (WINDOW, WINDOW), 1)
    rel = jj - ii + (WINDOW - 1)
    kk = jax.lax.broadcasted_iota(jnp.int32, (WINDOW, WINDOW, 128), 2)
    onehot = (rel[:, :, None] == kk).astype(jnp.float32)
    bias = jnp.sum(onehot * pos_ref[0][None, None, :], axis=2)

    outs = []
    dn_qk = (((2,), (2,)), ((0,), (0,)))
    dn_pv = (((2,), (1,)), ((0,), (0,)))
    for h in range(HEADS):
        sl = slice(h * HEAD_DIM, (h + 1) * HEAD_DIM)
        qh = qkv[:, sl].astype(jnp.bfloat16).reshape(J, WINDOW, HEAD_DIM)
        kh = qkv[:, INNER + h * HEAD_DIM:INNER + (h + 1) * HEAD_DIM]
        kh = kh.astype(jnp.bfloat16).reshape(J, WINDOW, HEAD_DIM)
        vh = qkv[:, 2 * INNER + h * HEAD_DIM:2 * INNER + (h + 1) * HEAD_DIM]
        vh = vh.astype(jnp.bfloat16).reshape(J, WINDOW, HEAD_DIM)
        dots = jax.lax.dot_general(qh, kh, dn_qk,
                                   preferred_element_type=jnp.float32)
        dots = dots * SCALE + bias[None, :, :]
        dots = dots - jnp.max(dots, axis=-1, keepdims=True)
        p = jnp.exp(dots)
        p = p / jnp.sum(p, axis=-1, keepdims=True)
        oh = jax.lax.dot_general(p.astype(jnp.bfloat16), vh, dn_pv,
                                 preferred_element_type=jnp.float32)
        outs.append(oh.reshape(T, HEAD_DIM).astype(jnp.bfloat16))

    attn_out = jnp.concatenate(outs, axis=1)
    out = jnp.dot(attn_out, wout_ref[...], preferred_element_type=jnp.float32)
    o_ref[0] = out + bout_ref[...]


def kernel(x, W_qkv, W_out, b_out, pos_embedding):
    b, p, f = x.shape
    x4 = x.reshape(b, WINDOW, NWIN, f).transpose(0, 2, 1, 3)
    x4 = x4.astype(jnp.bfloat16)
    wqkv = W_qkv.astype(jnp.bfloat16)
    wout = W_out.astype(jnp.bfloat16)
    bout = b_out.reshape(1, DIM)
    pos = jnp.pad(pos_embedding, (0, 1)).reshape(1, 128)

    grid = (b, NWIN // J)
    out = pl.pallas_call(
        _fused_kernel,
        grid=grid,
        in_specs=[
            pl.BlockSpec((1, J, WINDOW, DIM), lambda bi, ji: (bi, ji, 0, 0)),
            pl.BlockSpec((DIM, 3 * INNER), lambda bi, ji: (0, 0)),
            pl.BlockSpec((INNER, DIM), lambda bi, ji: (0, 0)),
            pl.BlockSpec((1, DIM), lambda bi, ji: (0, 0)),
            pl.BlockSpec((1, 128), lambda bi, ji: (0, 0)),
        ],
        out_specs=pl.BlockSpec((1, T, DIM), lambda bi, ji: (bi, ji, 0)),
        out_shape=jax.ShapeDtypeStruct((b, p, DIM), jnp.float32),
        compiler_params=pltpu.CompilerParams(
            dimension_semantics=("parallel", "arbitrary"),
        ),
    )(x4, wqkv, wout, bout, pos)
    return out
